# Initial kernel scaffold; baseline (speedup 1.0000x reference)
import jax
import jax.numpy as jnp
from jax.experimental import pallas as pl


def kernel(x, edge_index, cluster_index, cluster_id, W1, b1, Wfc, bfc):
    raise NotImplementedError("write your pallas kernel here")


# speedup vs baseline: 12.6073x; 12.6073x over previous
import functools
import jax
import jax.numpy as jnp
from jax import lax
from jax.experimental import pallas as pl
from jax.experimental.pallas import tpu as pltpu
from jax.experimental.pallas import tpu_sc as plsc

N = 10000
E = 160000
NFEAT = 256
NHID = 128
NCLASS = 10
CLUSTER = 100
NSEL = 5000

_MESH = plsc.VectorSubcoreMesh(core_axis_name="c", subcore_axis_name="s")
_NSC = 2
_NTILE = 16
_EB = 125
_NP = 10240
_RPT = _NP // _NTILE
_RC = 64
_DEGP = 10112
_SEL_PAD = 5120


def _scale_split_body(x_ref, deg_ref, deg1_ref, ga_ref, gb_ref, d1_ref):
    deg = deg_ref[...]
    dinv = lax.rsqrt(deg)
    x = x_ref[...]
    ga_ref[...] = dinv * x[:, :NHID]
    gb_ref[...] = dinv * x[:, NHID:]
    d1_ref[...] = lax.rsqrt(deg1_ref[...])


def _tc_scale_split(x_pad, deg_pad):
    return pl.pallas_call(
        _scale_split_body,
        out_shape=(
            jax.ShapeDtypeStruct((_NP, NHID), jnp.float32),
            jax.ShapeDtypeStruct((_NP, NHID), jnp.float32),
            jax.ShapeDtypeStruct((_NP,), jnp.float32),
        ),
    )(x_pad, deg_pad, deg_pad.reshape(_NP))


def _mid_scale_body(sa_ref, sb_ref, deg_ref, ga_ref, gb_ref):
    rdeg = 1.0 / deg_ref[...]
    ga_ref[...] = sa_ref[...] * rdeg
    gb_ref[...] = sb_ref[...] * rdeg


def _tc_mid_scale(sa, sb, deg_pad):
    return pl.pallas_call(
        _mid_scale_body,
        out_shape=(
            jax.ShapeDtypeStruct((_NP, NHID), jnp.float32),
            jax.ShapeDtypeStruct((_NP, NHID), jnp.float32),
        ),
    )(sa, sb, deg_pad)


def _tail_body(sa_ref, sb_ref, dsel_ref, cid_ref, w1a_ref, w1b_ref, b1_ref,
               wfa_ref, wfb_ref, bfc_ref, out_ref):
    dinv = dsel_ref[...]
    za = dinv * sa_ref[...]
    zb = dinv * sb_ref[...]
    hc = jnp.dot(za, w1a_ref[...].T, preferred_element_type=jnp.float32)
    hc += jnp.dot(zb, w1b_ref[...].T, preferred_element_type=jnp.float32)
    hc = jax.nn.relu(hc + b1_ref[...])

    cid = cid_ref[...]
    colsum = jnp.sum(cid, axis=0, keepdims=True)
    p = lax.dot_general(cid, hc, (((0,), (0,)), ((), ())),
                        preferred_element_type=jnp.float32)
    cf = p / colsum.T

    m = jnp.max(cid, axis=1, keepdims=True)
    iota = lax.broadcasted_iota(jnp.int32, (NSEL, CLUSTER), 1)
    cand = jnp.where(cid == m, iota, CLUSTER)
    amin = jnp.min(cand, axis=1, keepdims=True)
    onehot = (iota == amin).astype(jnp.float32)
    x1 = jnp.dot(onehot, cf, preferred_element_type=jnp.float32)

    wfa = wfa_ref[...]
    wfb = wfb_ref[...]
    bfc = bfc_ref[...]
    out_ref[:NSEL, :] = (
        jnp.dot(hc, wfa.T, preferred_element_type=jnp.float32)
        + jnp.dot(x1, wfb.T, preferred_element_type=jnp.float32) + bfc)
    out_ref[NSEL:, :] = (
        jnp.dot(x1, wfa.T, preferred_element_type=jnp.float32)
        + jnp.dot(hc, wfb.T, preferred_element_type=jnp.float32) + bfc)


def _tc_tail(sa_sel, sb_sel, d1_sel, cluster_id, W1, b1, Wfc, bfc):
    nclass2 = NCLASS * NCLASS
    return pl.pallas_call(
        _tail_body,
        out_shape=jax.ShapeDtypeStruct((2 * NSEL, nclass2), jnp.float32),
    )(sa_sel, sb_sel, d1_sel.reshape(NSEL, 1), cluster_id,
      W1[:, :NHID], W1[:, NHID:], b1.reshape(1, NHID),
      Wfc[:, :NHID], Wfc[:, NHID:], bfc.reshape(1, nclass2))


def _sc_degree_kernel(dst_rs, zeros_pad, ones_eb):
    nb = E // 32 // _EB

    @functools.partial(
        pl.kernel,
        out_type=jax.ShapeDtypeStruct((_NSC * _DEGP,), jnp.float32),
        mesh=_MESH,
        scratch_types=[
            pltpu.VMEM((1, nb, _EB), jnp.int32),
            pltpu.VMEM((_EB,), jnp.float32),
            pltpu.VMEM((_DEGP // _NTILE,), jnp.float32),
            pltpu.VMEM_SHARED((_DEGP,), jnp.float32),
            pltpu.SemaphoreType.DMA,
        ],
    )
    def k(dst_hbm, zeros_hbm, ones_hbm, degp_hbm, idx_v, ones_v, zbuf,
          shared_deg, sem):
        c = lax.axis_index("c")
        s = lax.axis_index("s")
        w = c * _NTILE + s
        seg = _DEGP // _NTILE
        pltpu.sync_copy(zeros_hbm.at[pl.ds(s * seg, seg)], zbuf)
        pltpu.sync_copy(zbuf, shared_deg.at[pl.ds(s * seg, seg)])
        pltpu.sync_copy(ones_hbm, ones_v)
        pltpu.sync_copy(dst_hbm.at[pl.ds(w, 1), :, :], idx_v)
        plsc.subcore_barrier()

        def body(j, carry):
            pltpu.sync_copy(ones_v, shared_deg.at[idx_v.at[0, j]], add=True)
            return carry

        lax.fori_loop(0, nb, body, 0)
        plsc.subcore_barrier()
        pltpu.sync_copy(shared_deg.at[pl.ds(s * seg, seg)], zbuf)
        pltpu.sync_copy(zbuf, degp_hbm.at[pl.ds(c * _DEGP + s * seg, seg)])

    return k(dst_rs, zeros_pad, ones_eb)


def _sc_degree(dst):
    dst_rs = dst.reshape(32, E // 32 // _EB, _EB)
    zeros_pad = jnp.zeros((_DEGP,), jnp.float32)
    ones_eb = jnp.ones((_EB,), jnp.float32)
    degp = _sc_degree_kernel(dst_rs, zeros_pad, ones_eb)
    return degp[:N] + degp[_DEGP:_DEGP + N]


def _sc_hop(ga, gb, src_rs, dst_rs):
    nb = E // _NTILE // _EB

    @functools.partial(
        pl.kernel,
        out_type=(jax.ShapeDtypeStruct((_NP, NHID), jnp.float32),
                  jax.ShapeDtypeStruct((_NP, NHID), jnp.float32)),
        mesh=_MESH,
        scratch_types=[
            pltpu.VMEM((1, nb, _EB), jnp.int32),
            pltpu.VMEM((1, nb, _EB), jnp.int32),
            pltpu.VMEM((_EB, NHID), jnp.float32),
            pltpu.VMEM_SHARED((_NP, NHID), jnp.float32),
            pltpu.SemaphoreType.DMA,
        ],
    )
    def k(ga_hbm, gb_hbm, src_hbm, dst_hbm, sa_hbm, sb_hbm,
          idxs_v, idxd_v, ebuf, accum, sem):
        cbuf = ebuf.at[pl.ds(0, _RC), :]
        c = lax.axis_index("c")
        s = lax.axis_index("s")
        r0 = s * _RPT

        def run_chunk(g_hbm, out_hbm):
            for j in range(_RPT // _RC):
                sl = pl.ds(r0 + j * _RC, _RC)
                pltpu.sync_copy(g_hbm.at[sl, :], cbuf)
                pltpu.sync_copy(cbuf, accum.at[sl, :])
            pltpu.sync_copy(src_hbm.at[pl.ds(s, 1), :, :], idxs_v)
            pltpu.sync_copy(dst_hbm.at[pl.ds(s, 1), :, :], idxd_v)
            plsc.subcore_barrier()

            def body(j, carry):
                pltpu.async_copy(g_hbm.at[idxs_v.at[0, j]], ebuf, sem).wait()
                pltpu.sync_copy(ebuf, accum.at[idxd_v.at[0, j]], add=True)
                return carry

            lax.fori_loop(0, nb, body, 0)
            plsc.subcore_barrier()
            for j in range(_RPT // _RC):
                sl = pl.ds(r0 + j * _RC, _RC)
                pltpu.sync_copy(accum.at[sl, :], cbuf)
                pltpu.sync_copy(cbuf, out_hbm.at[sl, :])

        @pl.when(c == 0)
        def _():
            run_chunk(ga_hbm, sa_hbm)

        @pl.when(c == 1)
        def _():
            run_chunk(gb_hbm, sb_hbm)

    return k(ga, gb, src_rs, dst_rs)


def _sc_gather_sel_kernel(sa, sb, d1, sel_rs):
    per_tile = _SEL_PAD // _NTILE
    nb = 4
    bs = per_tile // nb

    @functools.partial(
        pl.kernel,
        out_type=(jax.ShapeDtypeStruct((_SEL_PAD, NHID), jnp.float32),
                  jax.ShapeDtypeStruct((_SEL_PAD, NHID), jnp.float32),
                  jax.ShapeDtypeStruct((_SEL_PAD,), jnp.float32)),
        mesh=_MESH,
        scratch_types=[
            pltpu.VMEM((1, nb, bs), jnp.int32),
            pltpu.VMEM((bs, NHID), jnp.float32),
            pltpu.VMEM((bs,), jnp.float32),
            pltpu.SemaphoreType.DMA,
        ],
    )
    def k(sa_hbm, sb_hbm, d1_hbm, sel_hbm, oa_hbm, ob_hbm, o1_hbm,
          idx_v, buf, buf1, sem):
        c = lax.axis_index("c")
        s = lax.axis_index("s")
        pltpu.sync_copy(sel_hbm.at[pl.ds(s, 1), :, :], idx_v)

        def gather_chunk(src_hbm, out_hbm):
            for j in range(nb):
                base = s * per_tile + j * bs
                pltpu.async_copy(src_hbm.at[idx_v.at[0, j]], buf, sem).wait()
                pltpu.sync_copy(buf, out_hbm.at[pl.ds(base, bs), :])

        @pl.when(c == 0)
        def _():
            gather_chunk(sa_hbm, oa_hbm)
            for j in range(nb):
                base = s * per_tile + j * bs
                pltpu.async_copy(d1_hbm.at[idx_v.at[0, j]], buf1, sem).wait()
                pltpu.sync_copy(buf1, o1_hbm.at[pl.ds(base, bs)])

        @pl.when(c == 1)
        def _():
            gather_chunk(sb_hbm, ob_hbm)

    return k(sa, sb, d1, sel_rs)


def _sc_gather_sel(sa, sb, d1, sel):
    sel_pad = jnp.pad(sel, (0, _SEL_PAD - NSEL))
    sel_rs = sel_pad.reshape(_NTILE, 4, _SEL_PAD // 64)
    oa, ob, o1 = _sc_gather_sel_kernel(sa, sb, d1, sel_rs)
    return oa[:NSEL], ob[:NSEL], o1[:NSEL]


def kernel(x, edge_index, cluster_index, cluster_id, W1, b1, Wfc, bfc):
    src = edge_index[0].astype(jnp.int32)
    dst = edge_index[1].astype(jnp.int32)
    sel = cluster_index.astype(jnp.int32)
    src_rs = src.reshape(_NTILE, E // _NTILE // _EB, _EB)
    dst_rs = dst.reshape(_NTILE, E // _NTILE // _EB, _EB)

    hist = _sc_degree(dst)
    deg = hist + 1.0
    deg_pad = jnp.pad(deg, (0, _NP - N), constant_values=1.0).reshape(_NP, 1)
    x_pad = jnp.pad(x, ((0, _NP - N), (0, 0)))

    ga, gb, d1 = _tc_scale_split(x_pad, deg_pad)
    sa, sb = _sc_hop(ga, gb, src_rs, dst_rs)
    ga1, gb1 = _tc_mid_scale(sa, sb, deg_pad)
    sa2, sb2 = _sc_hop(ga1, gb1, src_rs, dst_rs)
    sa_sel, sb_sel, d1_sel = _sc_gather_sel(sa2, sb2, d1, sel)
    return _tc_tail(sa_sel, sb_sel, d1_sel, cluster_id, W1, b1, Wfc, bfc)


# speedup vs baseline: 18.4632x; 1.4645x over previous
import functools
import jax
import jax.numpy as jnp
from jax import lax
from jax.experimental import pallas as pl
from jax.experimental.pallas import tpu as pltpu
from jax.experimental.pallas import tpu_sc as plsc

N = 10000
E = 160000
NFEAT = 256
NHID = 128
NCLASS = 10
CLUSTER = 100
NSEL = 5000

_MESH = plsc.VectorSubcoreMesh(core_axis_name="c", subcore_axis_name="s")
_NSC = 2
_NTILE = 16
_EB = 125
_NP = 10240
_RPT = _NP // _NTILE
_RC = 64
_DEGP = 10112
_SEL_PAD = 5120


def _scale_split_body(x_ref, deg_ref, deg1_ref, ga_ref, gb_ref, d1_ref):
    deg = deg_ref[...]
    dinv = lax.rsqrt(deg)
    x = x_ref[...]
    ga_ref[...] = dinv * x[:, :NHID]
    gb_ref[...] = dinv * x[:, NHID:]
    d1_ref[...] = lax.rsqrt(deg1_ref[...])


def _tc_scale_split(x_pad, deg_pad):
    return pl.pallas_call(
        _scale_split_body,
        out_shape=(
            jax.ShapeDtypeStruct((_NP, NHID), jnp.float32),
            jax.ShapeDtypeStruct((_NP, NHID), jnp.float32),
            jax.ShapeDtypeStruct((_NP,), jnp.float32),
        ),
    )(x_pad, deg_pad, deg_pad.reshape(_NP))


def _mid_scale_body(sa_ref, sb_ref, deg_ref, ga_ref, gb_ref):
    rdeg = 1.0 / deg_ref[...]
    ga_ref[...] = sa_ref[...] * rdeg
    gb_ref[...] = sb_ref[...] * rdeg


def _tc_mid_scale(sa, sb, deg_pad):
    return pl.pallas_call(
        _mid_scale_body,
        out_shape=(
            jax.ShapeDtypeStruct((_NP, NHID), jnp.float32),
            jax.ShapeDtypeStruct((_NP, NHID), jnp.float32),
        ),
    )(sa, sb, deg_pad)


def _tail_body(sa_ref, sb_ref, dsel_ref, cid_ref, w1a_ref, w1b_ref, b1_ref,
               wfa_ref, wfb_ref, bfc_ref, out_ref):
    dinv = dsel_ref[...]
    za = dinv * sa_ref[...]
    zb = dinv * sb_ref[...]
    hc = jnp.dot(za, w1a_ref[...].T, preferred_element_type=jnp.float32)
    hc += jnp.dot(zb, w1b_ref[...].T, preferred_element_type=jnp.float32)
    hc = jax.nn.relu(hc + b1_ref[...])

    cid = cid_ref[...]
    colsum = jnp.sum(cid, axis=0, keepdims=True)
    p = lax.dot_general(cid, hc, (((0,), (0,)), ((), ())),
                        preferred_element_type=jnp.float32)
    cf = p / colsum.T

    m = jnp.max(cid, axis=1, keepdims=True)
    iota = lax.broadcasted_iota(jnp.int32, (NSEL, CLUSTER), 1)
    cand = jnp.where(cid == m, iota, CLUSTER)
    amin = jnp.min(cand, axis=1, keepdims=True)
    onehot = (iota == amin).astype(jnp.float32)
    x1 = jnp.dot(onehot, cf, preferred_element_type=jnp.float32)

    wfa = wfa_ref[...]
    wfb = wfb_ref[...]
    bfc = bfc_ref[...]
    out_ref[:NSEL, :] = (
        jnp.dot(hc, wfa.T, preferred_element_type=jnp.float32)
        + jnp.dot(x1, wfb.T, preferred_element_type=jnp.float32) + bfc)
    out_ref[NSEL:, :] = (
        jnp.dot(x1, wfa.T, preferred_element_type=jnp.float32)
        + jnp.dot(hc, wfb.T, preferred_element_type=jnp.float32) + bfc)


def _tc_tail(sa_sel, sb_sel, d1_sel, cluster_id, W1, b1, Wfc, bfc):
    nclass2 = NCLASS * NCLASS
    return pl.pallas_call(
        _tail_body,
        out_shape=jax.ShapeDtypeStruct((2 * NSEL, nclass2), jnp.float32),
    )(sa_sel, sb_sel, d1_sel.reshape(NSEL, 1), cluster_id,
      W1[:, :NHID], W1[:, NHID:], b1.reshape(1, NHID),
      Wfc[:, :NHID], Wfc[:, NHID:], bfc.reshape(1, nclass2))


def _sc_degree_kernel(dst_rs, zeros_pad, ones_eb):
    nb = E // 32 // _EB

    @functools.partial(
        pl.kernel,
        out_type=jax.ShapeDtypeStruct((_NSC * _DEGP,), jnp.float32),
        mesh=_MESH,
        scratch_types=[
            pltpu.VMEM((1, nb, _EB), jnp.int32),
            pltpu.VMEM((_EB,), jnp.float32),
            pltpu.VMEM((_DEGP // _NTILE,), jnp.float32),
            pltpu.VMEM_SHARED((_DEGP,), jnp.float32),
            pltpu.SemaphoreType.DMA,
        ],
    )
    def k(dst_hbm, zeros_hbm, ones_hbm, degp_hbm, idx_v, ones_v, zbuf,
          shared_deg, sem):
        c = lax.axis_index("c")
        s = lax.axis_index("s")
        w = c * _NTILE + s
        seg = _DEGP // _NTILE
        pltpu.sync_copy(zeros_hbm.at[pl.ds(s * seg, seg)], zbuf)
        pltpu.sync_copy(zbuf, shared_deg.at[pl.ds(s * seg, seg)])
        pltpu.sync_copy(ones_hbm, ones_v)
        pltpu.sync_copy(dst_hbm.at[pl.ds(w, 1), :, :], idx_v)
        plsc.subcore_barrier()

        def body(j, carry):
            pltpu.sync_copy(ones_v, shared_deg.at[idx_v.at[0, j]], add=True)
            return carry

        lax.fori_loop(0, nb, body, 0)
        plsc.subcore_barrier()
        pltpu.sync_copy(shared_deg.at[pl.ds(s * seg, seg)], zbuf)
        pltpu.sync_copy(zbuf, degp_hbm.at[pl.ds(c * _DEGP + s * seg, seg)])

    return k(dst_rs, zeros_pad, ones_eb)


def _sc_degree(dst):
    dst_rs = dst.reshape(32, E // 32 // _EB, _EB)
    zeros_pad = jnp.zeros((_DEGP,), jnp.float32)
    ones_eb = jnp.ones((_EB,), jnp.float32)
    degp = _sc_degree_kernel(dst_rs, zeros_pad, ones_eb)
    return degp[:N] + degp[_DEGP:_DEGP + N]


_NB = E // _NTILE // _EB
_NBH = _NB // 2
_SB = _SEL_PAD // 64


def _hop_edges(g_hbm, src_hbm, dst_hbm, accum, idxs_v, idxd_v,
               ebuf0, ebuf1, sem0, sem1, s):
    for h in range(2):
        hs = pl.ds(h * _NBH, _NBH)
        pltpu.sync_copy(src_hbm.at[pl.ds(s, 1), hs, :], idxs_v)
        pltpu.sync_copy(dst_hbm.at[pl.ds(s, 1), hs, :], idxd_v)
        pltpu.async_copy(g_hbm.at[idxs_v.at[0, 0]], ebuf0, sem0)
        pltpu.async_copy(g_hbm.at[idxs_v.at[0, 1]], ebuf1, sem1)

        def body(i, carry):
            j0 = 2 * i
            pltpu.make_async_copy(g_hbm.at[idxs_v.at[0, j0]], ebuf0,
                                  sem0).wait()
            pltpu.sync_copy(ebuf0, accum.at[idxd_v.at[0, j0]], add=True)

            @pl.when(j0 + 2 < _NBH)
            def _():
                pltpu.async_copy(g_hbm.at[idxs_v.at[0, j0 + 2]], ebuf0, sem0)

            j1 = j0 + 1
            pltpu.make_async_copy(g_hbm.at[idxs_v.at[0, j1]], ebuf1,
                                  sem1).wait()
            pltpu.sync_copy(ebuf1, accum.at[idxd_v.at[0, j1]], add=True)

            @pl.when(j1 + 2 < _NBH)
            def _():
                pltpu.async_copy(g_hbm.at[idxs_v.at[0, j1 + 2]], ebuf1, sem1)

            return carry

        lax.fori_loop(0, _NBH // 2, body, 0)


def _hop_init(g_hbm, accum, cbuf, s):
    for j in range(_RPT // _RC):
        sl = pl.ds(s * _RPT + j * _RC, _RC)
        pltpu.sync_copy(g_hbm.at[sl, :], cbuf)
        pltpu.sync_copy(cbuf, accum.at[sl, :])


_HOP_SCRATCH = [
    pltpu.VMEM((1, _NBH, _EB), jnp.int32),
    pltpu.VMEM((1, _NBH, _EB), jnp.int32),
    pltpu.VMEM((_EB, NHID), jnp.float32),
    pltpu.VMEM((_EB, NHID), jnp.float32),
    pltpu.VMEM_SHARED((_NP, NHID), jnp.float32),
    pltpu.SemaphoreType.DMA,
    pltpu.SemaphoreType.DMA,
]


def _sc_hop(ga, gb, src_rs, dst_rs):
    @functools.partial(
        pl.kernel,
        out_type=(jax.ShapeDtypeStruct((_NP, NHID), jnp.float32),
                  jax.ShapeDtypeStruct((_NP, NHID), jnp.float32)),
        mesh=_MESH,
        scratch_types=_HOP_SCRATCH,
    )
    def k(ga_hbm, gb_hbm, src_hbm, dst_hbm, sa_hbm, sb_hbm,
          idxs_v, idxd_v, ebuf0, ebuf1, accum, sem0, sem1):
        cbuf = ebuf0.at[pl.ds(0, _RC), :]
        c = lax.axis_index("c")
        s = lax.axis_index("s")

        def run_chunk(g_hbm, out_hbm):
            _hop_init(g_hbm, accum, cbuf, s)
            plsc.subcore_barrier()
            _hop_edges(g_hbm, src_hbm, dst_hbm, accum, idxs_v, idxd_v,
                       ebuf0, ebuf1, sem0, sem1, s)
            plsc.subcore_barrier()
            for j in range(_RPT // _RC):
                sl = pl.ds(s * _RPT + j * _RC, _RC)
                pltpu.sync_copy(accum.at[sl, :], cbuf)
                pltpu.sync_copy(cbuf, out_hbm.at[sl, :])

        @pl.when(c == 0)
        def _():
            run_chunk(ga_hbm, sa_hbm)

        @pl.when(c == 1)
        def _():
            run_chunk(gb_hbm, sb_hbm)

    return k(ga, gb, src_rs, dst_rs)


def _sc_hop_final(ga, gb, src_rs, dst_rs, d1, sel_rs):
    @functools.partial(
        pl.kernel,
        out_type=(jax.ShapeDtypeStruct((_SEL_PAD, NHID), jnp.float32),
                  jax.ShapeDtypeStruct((_SEL_PAD, NHID), jnp.float32),
                  jax.ShapeDtypeStruct((_SEL_PAD,), jnp.float32)),
        mesh=_MESH,
        scratch_types=_HOP_SCRATCH + [
            pltpu.VMEM((1, 4, _SB), jnp.int32),
            pltpu.VMEM((_SB,), jnp.float32),
        ],
    )
    def k(ga_hbm, gb_hbm, src_hbm, dst_hbm, d1_hbm, sel_hbm,
          oa_hbm, ob_hbm, o1_hbm,
          idxs_v, idxd_v, ebuf0, ebuf1, accum, sem0, sem1, sel_v, dbuf):
        cbuf = ebuf0.at[pl.ds(0, _RC), :]
        sbuf = ebuf0.at[pl.ds(0, _SB), :]
        c = lax.axis_index("c")
        s = lax.axis_index("s")

        def run_chunk(g_hbm, out_hbm):
            _hop_init(g_hbm, accum, cbuf, s)
            plsc.subcore_barrier()
            _hop_edges(g_hbm, src_hbm, dst_hbm, accum, idxs_v, idxd_v,
                       ebuf0, ebuf1, sem0, sem1, s)
            plsc.subcore_barrier()
            pltpu.sync_copy(sel_hbm.at[pl.ds(s, 1), :, :], sel_v)
            for j in range(4):
                base = s * (_SEL_PAD // _NTILE) + j * _SB
                pltpu.async_copy(accum.at[sel_v.at[0, j]], sbuf, sem0).wait()
                pltpu.sync_copy(sbuf, out_hbm.at[pl.ds(base, _SB), :])

        @pl.when(c == 0)
        def _():
            run_chunk(ga_hbm, oa_hbm)
            for j in range(4):
                base = s * (_SEL_PAD // _NTILE) + j * _SB
                pltpu.async_copy(d1_hbm.at[sel_v.at[0, j]], dbuf, sem0).wait()
                pltpu.sync_copy(dbuf, o1_hbm.at[pl.ds(base, _SB)])

        @pl.when(c == 1)
        def _():
            run_chunk(gb_hbm, ob_hbm)

    return k(ga, gb, src_rs, dst_rs, d1, sel_rs)


def kernel(x, edge_index, cluster_index, cluster_id, W1, b1, Wfc, bfc):
    src = edge_index[0].astype(jnp.int32)
    dst = edge_index[1].astype(jnp.int32)
    sel = cluster_index.astype(jnp.int32)
    src_rs = src.reshape(_NTILE, E // _NTILE // _EB, _EB)
    dst_rs = dst.reshape(_NTILE, E // _NTILE // _EB, _EB)

    hist = _sc_degree(dst)
    deg = hist + 1.0
    deg_pad = jnp.pad(deg, (0, _NP - N), constant_values=1.0).reshape(_NP, 1)
    x_pad = jnp.pad(x, ((0, _NP - N), (0, 0)))

    sel_pad = jnp.pad(sel, (0, _SEL_PAD - NSEL))
    sel_rs = sel_pad.reshape(_NTILE, 4, _SB)

    ga, gb, d1 = _tc_scale_split(x_pad, deg_pad)
    sa, sb = _sc_hop(ga, gb, src_rs, dst_rs)
    ga1, gb1 = _tc_mid_scale(sa, sb, deg_pad)
    oa, ob, o1 = _sc_hop_final(ga1, gb1, src_rs, dst_rs, d1, sel_rs)
    return _tc_tail(oa[:NSEL], ob[:NSEL], o1[:NSEL], cluster_id,
                    W1, b1, Wfc, bfc)


# speedup vs baseline: 18.7894x; 1.0177x over previous
import functools
import jax
import jax.numpy as jnp
from jax import lax
from jax.experimental import pallas as pl
from jax.experimental.pallas import tpu as pltpu
from jax.experimental.pallas import tpu_sc as plsc

N = 10000
E = 160000
NFEAT = 256
NHID = 128
NCLASS = 10
CLUSTER = 100
NSEL = 5000

_MESH = plsc.VectorSubcoreMesh(core_axis_name="c", subcore_axis_name="s")
_NSC = 2
_NTILE = 16
_EB = 125
_NP = 10240
_RPT = _NP // _NTILE
_RC = 64
_DEGP = 10112
_SEL_PAD = 5120


def _scale_split_body(x_ref, deg_ref, deg1_ref, ga_ref, gb_ref, d1_ref):
    deg = deg_ref[...]
    dinv = lax.rsqrt(deg)
    x = x_ref[...]
    ga_ref[:N, :] = dinv * x[:, :NHID]
    gb_ref[:N, :] = dinv * x[:, NHID:]
    ga_ref[N:, :] = jnp.zeros((_NP - N, NHID), jnp.float32)
    gb_ref[N:, :] = jnp.zeros((_NP - N, NHID), jnp.float32)
    d1_ref[:N] = lax.rsqrt(deg1_ref[...])
    d1_ref[N:] = jnp.ones((_NP - N,), jnp.float32)


def _tc_scale_split(x, deg):
    return pl.pallas_call(
        _scale_split_body,
        out_shape=(
            jax.ShapeDtypeStruct((_NP, NHID), jnp.float32),
            jax.ShapeDtypeStruct((_NP, NHID), jnp.float32),
            jax.ShapeDtypeStruct((_NP,), jnp.float32),
        ),
    )(x, deg.reshape(N, 1), deg)


def _mid_scale_body(sa_ref, sb_ref, deg_ref, ga_ref, gb_ref):
    rdeg = 1.0 / deg_ref[...]
    ga_ref[...] = sa_ref[...] * rdeg
    gb_ref[...] = sb_ref[...] * rdeg


def _tc_mid_scale(sa, sb, deg_pad):
    return pl.pallas_call(
        _mid_scale_body,
        out_shape=(
            jax.ShapeDtypeStruct((_NP, NHID), jnp.float32),
            jax.ShapeDtypeStruct((_NP, NHID), jnp.float32),
        ),
    )(sa, sb, deg_pad)


def _tail_body(sa_ref, sb_ref, dsel_ref, cid_ref, w1a_ref, w1b_ref, b1_ref,
               wfa_ref, wfb_ref, bfc_ref, out_ref):
    dinv = dsel_ref[:NSEL, :]
    za = dinv * sa_ref[:NSEL, :]
    zb = dinv * sb_ref[:NSEL, :]
    hc = jnp.dot(za, w1a_ref[...].T, preferred_element_type=jnp.float32)
    hc += jnp.dot(zb, w1b_ref[...].T, preferred_element_type=jnp.float32)
    hc = jax.nn.relu(hc + b1_ref[...])

    cid = cid_ref[...]
    colsum = jnp.sum(cid, axis=0, keepdims=True)
    p = lax.dot_general(cid, hc, (((0,), (0,)), ((), ())),
                        preferred_element_type=jnp.float32)
    cf = p / colsum.T

    m = jnp.max(cid, axis=1, keepdims=True)
    iota = lax.broadcasted_iota(jnp.int32, (NSEL, CLUSTER), 1)
    cand = jnp.where(cid == m, iota, CLUSTER)
    amin = jnp.min(cand, axis=1, keepdims=True)
    onehot = (iota == amin).astype(jnp.float32)
    x1 = jnp.dot(onehot, cf, preferred_element_type=jnp.float32)

    wfa = wfa_ref[...]
    wfb = wfb_ref[...]
    bfc = bfc_ref[...]
    out_ref[:NSEL, :] = (
        jnp.dot(hc, wfa.T, preferred_element_type=jnp.float32)
        + jnp.dot(x1, wfb.T, preferred_element_type=jnp.float32) + bfc)
    out_ref[NSEL:, :] = (
        jnp.dot(x1, wfa.T, preferred_element_type=jnp.float32)
        + jnp.dot(hc, wfb.T, preferred_element_type=jnp.float32) + bfc)


def _tc_tail(sa_pad, sb_pad, d1_pad, cluster_id, W1, b1, Wfc, bfc):
    nclass2 = NCLASS * NCLASS
    return pl.pallas_call(
        _tail_body,
        out_shape=jax.ShapeDtypeStruct((2 * NSEL, nclass2), jnp.float32),
    )(sa_pad, sb_pad, d1_pad.reshape(_SEL_PAD, 1), cluster_id,
      W1[:, :NHID], W1[:, NHID:], b1.reshape(1, NHID),
      Wfc[:, :NHID], Wfc[:, NHID:], bfc.reshape(1, nclass2))


def _sc_degree_kernel(dst_rs, zeros_pad, ones_eb):
    nb = E // 32 // _EB

    @functools.partial(
        pl.kernel,
        out_type=jax.ShapeDtypeStruct((_NSC * _DEGP,), jnp.float32),
        mesh=_MESH,
        scratch_types=[
            pltpu.VMEM((1, nb, _EB), jnp.int32),
            pltpu.VMEM((_EB,), jnp.float32),
            pltpu.VMEM((_DEGP // _NTILE,), jnp.float32),
            pltpu.VMEM_SHARED((_DEGP,), jnp.float32),
            pltpu.SemaphoreType.DMA,
        ],
    )
    def k(dst_hbm, zeros_hbm, ones_hbm, degp_hbm, idx_v, ones_v, zbuf,
          shared_deg, sem):
        c = lax.axis_index("c")
        s = lax.axis_index("s")
        w = c * _NTILE + s
        seg = _DEGP // _NTILE
        pltpu.sync_copy(zeros_hbm.at[pl.ds(s * seg, seg)], zbuf)
        pltpu.sync_copy(zbuf, shared_deg.at[pl.ds(s * seg, seg)])
        pltpu.sync_copy(ones_hbm, ones_v)
        pltpu.sync_copy(dst_hbm.at[pl.ds(w, 1), :, :], idx_v)
        plsc.subcore_barrier()

        def body(j, carry):
            pltpu.sync_copy(ones_v, shared_deg.at[idx_v.at[0, j]], add=True)
            return carry

        lax.fori_loop(0, nb, body, 0)
        plsc.subcore_barrier()
        pltpu.sync_copy(shared_deg.at[pl.ds(s * seg, seg)], zbuf)
        pltpu.sync_copy(zbuf, degp_hbm.at[pl.ds(c * _DEGP + s * seg, seg)])

    return k(dst_rs, zeros_pad, ones_eb)


def _sc_degree(dst):
    dst_rs = dst.reshape(32, E // 32 // _EB, _EB)
    zeros_pad = jnp.zeros((_DEGP,), jnp.float32)
    ones_eb = jnp.ones((_EB,), jnp.float32)
    degp = _sc_degree_kernel(dst_rs, zeros_pad, ones_eb)
    return degp[:N] + degp[_DEGP:_DEGP + N]


_NB = E // _NTILE // _EB
_NBH = _NB // 2
_SB = _SEL_PAD // 64


def _hop_edges(g_hbm, src_hbm, dst_hbm, accum, idxs_v, idxd_v,
               ebuf0, ebuf1, sem0, sem1, s):
    for h in range(2):
        hs = pl.ds(h * _NBH, _NBH)
        pltpu.sync_copy(src_hbm.at[pl.ds(s, 1), hs, :], idxs_v)
        pltpu.sync_copy(dst_hbm.at[pl.ds(s, 1), hs, :], idxd_v)
        pltpu.async_copy(g_hbm.at[idxs_v.at[0, 0]], ebuf0, sem0)
        pltpu.async_copy(g_hbm.at[idxs_v.at[0, 1]], ebuf1, sem1)

        def body(i, carry):
            j0 = 2 * i
            pltpu.make_async_copy(g_hbm.at[idxs_v.at[0, j0]], ebuf0,
                                  sem0).wait()
            pltpu.sync_copy(ebuf0, accum.at[idxd_v.at[0, j0]], add=True)

            @pl.when(j0 + 2 < _NBH)
            def _():
                pltpu.async_copy(g_hbm.at[idxs_v.at[0, j0 + 2]], ebuf0, sem0)

            j1 = j0 + 1
            pltpu.make_async_copy(g_hbm.at[idxs_v.at[0, j1]], ebuf1,
                                  sem1).wait()
            pltpu.sync_copy(ebuf1, accum.at[idxd_v.at[0, j1]], add=True)

            @pl.when(j1 + 2 < _NBH)
            def _():
                pltpu.async_copy(g_hbm.at[idxs_v.at[0, j1 + 2]], ebuf1, sem1)

            return carry

        lax.fori_loop(0, _NBH // 2, body, 0)


def _hop_init(g_hbm, accum, cbuf, s):
    for j in range(_RPT // _RC):
        sl = pl.ds(s * _RPT + j * _RC, _RC)
        pltpu.sync_copy(g_hbm.at[sl, :], cbuf)
        pltpu.sync_copy(cbuf, accum.at[sl, :])


_HOP_SCRATCH = [
    pltpu.VMEM((1, _NBH, _EB), jnp.int32),
    pltpu.VMEM((1, _NBH, _EB), jnp.int32),
    pltpu.VMEM((_EB, NHID), jnp.float32),
    pltpu.VMEM((_EB, NHID), jnp.float32),
    pltpu.VMEM_SHARED((_NP, NHID), jnp.float32),
    pltpu.SemaphoreType.DMA,
    pltpu.SemaphoreType.DMA,
]


def _sc_hop(ga, gb, src_rs, dst_rs):
    @functools.partial(
        pl.kernel,
        out_type=(jax.ShapeDtypeStruct((_NP, NHID), jnp.float32),
                  jax.ShapeDtypeStruct((_NP, NHID), jnp.float32)),
        mesh=_MESH,
        scratch_types=_HOP_SCRATCH,
    )
    def k(ga_hbm, gb_hbm, src_hbm, dst_hbm, sa_hbm, sb_hbm,
          idxs_v, idxd_v, ebuf0, ebuf1, accum, sem0, sem1):
        cbuf = ebuf0.at[pl.ds(0, _RC), :]
        c = lax.axis_index("c")
        s = lax.axis_index("s")

        def run_chunk(g_hbm, out_hbm):
            _hop_init(g_hbm, accum, cbuf, s)
            plsc.subcore_barrier()
            _hop_edges(g_hbm, src_hbm, dst_hbm, accum, idxs_v, idxd_v,
                       ebuf0, ebuf1, sem0, sem1, s)
            plsc.subcore_barrier()
            for j in range(_RPT // _RC):
                sl = pl.ds(s * _RPT + j * _RC, _RC)
                pltpu.sync_copy(accum.at[sl, :], cbuf)
                pltpu.sync_copy(cbuf, out_hbm.at[sl, :])

        @pl.when(c == 0)
        def _():
            run_chunk(ga_hbm, sa_hbm)

        @pl.when(c == 1)
        def _():
            run_chunk(gb_hbm, sb_hbm)

    return k(ga, gb, src_rs, dst_rs)


def _sc_hop_final(ga, gb, src_rs, dst_rs, d1, sel_rs):
    @functools.partial(
        pl.kernel,
        out_type=(jax.ShapeDtypeStruct((_SEL_PAD, NHID), jnp.float32),
                  jax.ShapeDtypeStruct((_SEL_PAD, NHID), jnp.float32),
                  jax.ShapeDtypeStruct((_SEL_PAD,), jnp.float32)),
        mesh=_MESH,
        scratch_types=_HOP_SCRATCH + [
            pltpu.VMEM((1, 4, _SB), jnp.int32),
            pltpu.VMEM((_SB,), jnp.float32),
        ],
    )
    def k(ga_hbm, gb_hbm, src_hbm, dst_hbm, d1_hbm, sel_hbm,
          oa_hbm, ob_hbm, o1_hbm,
          idxs_v, idxd_v, ebuf0, ebuf1, accum, sem0, sem1, sel_v, dbuf):
        cbuf = ebuf0.at[pl.ds(0, _RC), :]
        sbuf = ebuf0.at[pl.ds(0, _SB), :]
        c = lax.axis_index("c")
        s = lax.axis_index("s")

        def run_chunk(g_hbm, out_hbm):
            _hop_init(g_hbm, accum, cbuf, s)
            plsc.subcore_barrier()
            _hop_edges(g_hbm, src_hbm, dst_hbm, accum, idxs_v, idxd_v,
                       ebuf0, ebuf1, sem0, sem1, s)
            plsc.subcore_barrier()
            pltpu.sync_copy(sel_hbm.at[pl.ds(s, 1), :, :], sel_v)
            for j in range(4):
                base = s * (_SEL_PAD // _NTILE) + j * _SB
                pltpu.async_copy(accum.at[sel_v.at[0, j]], sbuf, sem0).wait()
                pltpu.sync_copy(sbuf, out_hbm.at[pl.ds(base, _SB), :])

        @pl.when(c == 0)
        def _():
            run_chunk(ga_hbm, oa_hbm)
            for j in range(4):
                base = s * (_SEL_PAD // _NTILE) + j * _SB
                pltpu.async_copy(d1_hbm.at[sel_v.at[0, j]], dbuf, sem0).wait()
                pltpu.sync_copy(dbuf, o1_hbm.at[pl.ds(base, _SB)])

        @pl.when(c == 1)
        def _():
            run_chunk(gb_hbm, ob_hbm)

    return k(ga, gb, src_rs, dst_rs, d1, sel_rs)


def kernel(x, edge_index, cluster_index, cluster_id, W1, b1, Wfc, bfc):
    src = edge_index[0].astype(jnp.int32)
    dst = edge_index[1].astype(jnp.int32)
    sel = cluster_index.astype(jnp.int32)
    src_rs = src.reshape(_NTILE, E // _NTILE // _EB, _EB)
    dst_rs = dst.reshape(_NTILE, E // _NTILE // _EB, _EB)

    hist = _sc_degree(dst)
    deg = hist + 1.0
    deg_pad = jnp.pad(deg, (0, _NP - N), constant_values=1.0).reshape(_NP, 1)

    sel_pad = jnp.pad(sel, (0, _SEL_PAD - NSEL))
    sel_rs = sel_pad.reshape(_NTILE, 4, _SB)

    ga, gb, d1 = _tc_scale_split(x, deg)
    sa, sb = _sc_hop(ga, gb, src_rs, dst_rs)
    ga1, gb1 = _tc_mid_scale(sa, sb, deg_pad)
    oa, ob, o1 = _sc_hop_final(ga1, gb1, src_rs, dst_rs, d1, sel_rs)
    return _tc_tail(oa, ob, o1, cluster_id, W1, b1, Wfc, bfc)


# speedup vs baseline: 19.0577x; 1.0143x over previous
import functools
import jax
import jax.numpy as jnp
from jax import lax
from jax.experimental import pallas as pl
from jax.experimental.pallas import tpu as pltpu
from jax.experimental.pallas import tpu_sc as plsc

N = 10000
E = 160000
NFEAT = 256
NHID = 128
NCLASS = 10
CLUSTER = 100
NSEL = 5000

_MESH = plsc.VectorSubcoreMesh(core_axis_name="c", subcore_axis_name="s")
_NSC = 2
_NTILE = 16
_EB = 125
_NP = 10240
_RPT = _NP // _NTILE
_RC = 64
_DEGP = 10112
_SEL_PAD = 5120


def _scale_split_body(x_ref, deg_ref, deg1_ref, ga_ref, gb_ref, d1_ref):
    deg = deg_ref[...]
    dinv = lax.rsqrt(deg)
    x = x_ref[...]
    ga_ref[:N, :] = dinv * x[:, :NHID]
    gb_ref[:N, :] = dinv * x[:, NHID:]
    ga_ref[N:, :] = jnp.zeros((_NP - N, NHID), jnp.float32)
    gb_ref[N:, :] = jnp.zeros((_NP - N, NHID), jnp.float32)
    d1_ref[:N] = lax.rsqrt(deg1_ref[...])
    d1_ref[N:] = jnp.ones((_NP - N,), jnp.float32)


def _tc_scale_split(x, deg):
    return pl.pallas_call(
        _scale_split_body,
        out_shape=(
            jax.ShapeDtypeStruct((_NP, NHID), jnp.float32),
            jax.ShapeDtypeStruct((_NP, NHID), jnp.float32),
            jax.ShapeDtypeStruct((_NP,), jnp.float32),
        ),
    )(x, deg.reshape(N, 1), deg)


def _mid_scale_body(sa_ref, sb_ref, deg_ref, ga_ref, gb_ref):
    rdeg = 1.0 / deg_ref[...]
    ga_ref[...] = sa_ref[...] * rdeg
    gb_ref[...] = sb_ref[...] * rdeg


def _tc_mid_scale(sa, sb, deg_pad):
    return pl.pallas_call(
        _mid_scale_body,
        out_shape=(
            jax.ShapeDtypeStruct((_NP, NHID), jnp.float32),
            jax.ShapeDtypeStruct((_NP, NHID), jnp.float32),
        ),
    )(sa, sb, deg_pad)


def _tail_body(sa_ref, sb_ref, dsel_ref, cid_ref, w1a_ref, w1b_ref, b1_ref,
               wfa_ref, wfb_ref, bfc_ref, out_ref):
    dinv = dsel_ref[:NSEL, :]
    za = dinv * sa_ref[:NSEL, :]
    zb = dinv * sb_ref[:NSEL, :]
    hc = jnp.dot(za, w1a_ref[...].T, preferred_element_type=jnp.float32)
    hc += jnp.dot(zb, w1b_ref[...].T, preferred_element_type=jnp.float32)
    hc = jax.nn.relu(hc + b1_ref[...])

    cid = cid_ref[...]
    colsum = jnp.sum(cid, axis=0, keepdims=True)
    p = lax.dot_general(cid, hc, (((0,), (0,)), ((), ())),
                        preferred_element_type=jnp.float32)
    cf = p / colsum.T

    m = jnp.max(cid, axis=1, keepdims=True)
    iota = lax.broadcasted_iota(jnp.int32, (NSEL, CLUSTER), 1)
    cand = jnp.where(cid == m, iota, CLUSTER)
    amin = jnp.min(cand, axis=1, keepdims=True)
    onehot = (iota == amin).astype(jnp.float32)
    x1 = jnp.dot(onehot, cf, preferred_element_type=jnp.float32)

    wfa = wfa_ref[...]
    wfb = wfb_ref[...]
    bfc = bfc_ref[...]
    out_ref[:NSEL, :] = (
        jnp.dot(hc, wfa.T, preferred_element_type=jnp.float32)
        + jnp.dot(x1, wfb.T, preferred_element_type=jnp.float32) + bfc)
    out_ref[NSEL:, :] = (
        jnp.dot(x1, wfa.T, preferred_element_type=jnp.float32)
        + jnp.dot(hc, wfb.T, preferred_element_type=jnp.float32) + bfc)


def _tc_tail(sa_pad, sb_pad, d1_pad, cluster_id, W1, b1, Wfc, bfc):
    nclass2 = NCLASS * NCLASS
    return pl.pallas_call(
        _tail_body,
        out_shape=jax.ShapeDtypeStruct((2 * NSEL, nclass2), jnp.float32),
    )(sa_pad, sb_pad, d1_pad.reshape(_SEL_PAD, 1), cluster_id,
      W1[:, :NHID], W1[:, NHID:], b1.reshape(1, NHID),
      Wfc[:, :NHID], Wfc[:, NHID:], bfc.reshape(1, nclass2))


def _sc_degree_kernel(dst_rs, zeros_pad, ones_eb):
    nb = E // 32 // _EB

    @functools.partial(
        pl.kernel,
        out_type=jax.ShapeDtypeStruct((_NSC * _DEGP,), jnp.float32),
        mesh=_MESH,
        scratch_types=[
            pltpu.VMEM((1, nb, _EB), jnp.int32),
            pltpu.VMEM((_EB,), jnp.float32),
            pltpu.VMEM((_DEGP // _NTILE,), jnp.float32),
            pltpu.VMEM_SHARED((_DEGP,), jnp.float32),
            pltpu.SemaphoreType.DMA,
        ],
    )
    def k(dst_hbm, zeros_hbm, ones_hbm, degp_hbm, idx_v, ones_v, zbuf,
          shared_deg, sem):
        c = lax.axis_index("c")
        s = lax.axis_index("s")
        w = c * _NTILE + s
        seg = _DEGP // _NTILE
        pltpu.sync_copy(zeros_hbm.at[pl.ds(s * seg, seg)], zbuf)
        pltpu.sync_copy(zbuf, shared_deg.at[pl.ds(s * seg, seg)])
        pltpu.sync_copy(ones_hbm, ones_v)
        pltpu.sync_copy(dst_hbm.at[pl.ds(w, 1), :, :], idx_v)
        plsc.subcore_barrier()

        def body(j, carry):
            pltpu.sync_copy(ones_v, shared_deg.at[idx_v.at[0, j]], add=True)
            return carry

        lax.fori_loop(0, nb, body, 0)
        plsc.subcore_barrier()
        pltpu.sync_copy(shared_deg.at[pl.ds(s * seg, seg)], zbuf)
        pltpu.sync_copy(zbuf, degp_hbm.at[pl.ds(c * _DEGP + s * seg, seg)])

    return k(dst_rs, zeros_pad, ones_eb)


def _sc_degree(dst):
    dst_rs = dst.reshape(32, E // 32 // _EB, _EB)
    zeros_pad = jnp.zeros((_DEGP,), jnp.float32)
    ones_eb = jnp.ones((_EB,), jnp.float32)
    degp = _sc_degree_kernel(dst_rs, zeros_pad, ones_eb)
    return degp[:N] + degp[_DEGP:_DEGP + N]


_NB = E // _NTILE // _EB
_NBH = _NB // 2
_SB = _SEL_PAD // 64


def _hop_edges(g_hbm, src_hbm, dst_hbm, accum, idxs_v, idxd_v,
               ebuf0, ebuf1, sem0, sem1, s):
    for h in range(2):
        hs = pl.ds(h * _NBH, _NBH)
        pltpu.sync_copy(src_hbm.at[pl.ds(s, 1), hs, :], idxs_v)
        pltpu.sync_copy(dst_hbm.at[pl.ds(s, 1), hs, :], idxd_v)
        pltpu.async_copy(g_hbm.at[idxs_v.at[0, 0]], ebuf0, sem0)
        pltpu.async_copy(g_hbm.at[idxs_v.at[0, 1]], ebuf1, sem1)

        def body(i, carry):
            j0 = 2 * i
            pltpu.make_async_copy(g_hbm.at[idxs_v.at[0, j0]], ebuf0,
                                  sem0).wait()
            pltpu.sync_copy(ebuf0, accum.at[idxd_v.at[0, j0]], add=True)

            @pl.when(j0 + 2 < _NBH)
            def _():
                pltpu.async_copy(g_hbm.at[idxs_v.at[0, j0 + 2]], ebuf0, sem0)

            j1 = j0 + 1
            pltpu.make_async_copy(g_hbm.at[idxs_v.at[0, j1]], ebuf1,
                                  sem1).wait()
            pltpu.sync_copy(ebuf1, accum.at[idxd_v.at[0, j1]], add=True)

            @pl.when(j1 + 2 < _NBH)
            def _():
                pltpu.async_copy(g_hbm.at[idxs_v.at[0, j1 + 2]], ebuf1, sem1)

            return carry

        lax.fori_loop(0, _NBH // 2, body, 0)


def _hop_init(g_hbm, accum, cbuf, s):
    for j in range(_RPT // _RC):
        sl = pl.ds(s * _RPT + j * _RC, _RC)
        pltpu.sync_copy(g_hbm.at[sl, :], cbuf)
        pltpu.sync_copy(cbuf, accum.at[sl, :])


_HOP_SCRATCH = [
    pltpu.VMEM((1, _NBH, _EB), jnp.int32),
    pltpu.VMEM((1, _NBH, _EB), jnp.int32),
    pltpu.VMEM((_EB, NHID), jnp.float32),
    pltpu.VMEM((_EB, NHID), jnp.float32),
    pltpu.VMEM_SHARED((_NP, NHID), jnp.float32),
    pltpu.SemaphoreType.DMA,
    pltpu.SemaphoreType.DMA,
]


def _sc_hop(ga, gb, src_rs, dst_rs, deg1d):
    @functools.partial(
        pl.kernel,
        out_type=(jax.ShapeDtypeStruct((_NP, NHID), jnp.float32),
                  jax.ShapeDtypeStruct((_NP, NHID), jnp.float32)),
        mesh=_MESH,
        scratch_types=_HOP_SCRATCH + [
            pltpu.VMEM((_RPT,), jnp.float32),
        ],
    )
    def k(ga_hbm, gb_hbm, src_hbm, dst_hbm, deg_hbm, sa_hbm, sb_hbm,
          idxs_v, idxd_v, ebuf0, ebuf1, accum, sem0, sem1, degv):
        c = lax.axis_index("c")
        s = lax.axis_index("s")

        def run_chunk(g_hbm, out_hbm):
            cbuf = ebuf0.at[pl.ds(0, _RC), :]
            _hop_init(g_hbm, accum, cbuf, s)
            pltpu.sync_copy(deg_hbm.at[pl.ds(s * _RPT, _RPT)], degv)
            plsc.subcore_barrier()
            _hop_edges(g_hbm, src_hbm, dst_hbm, accum, idxs_v, idxd_v,
                       ebuf0, ebuf1, sem0, sem1, s)
            plsc.subcore_barrier()

            def wout(j, carry):
                sl = pl.ds(s * _RPT + j * _RC, _RC)
                pltpu.sync_copy(accum.at[sl, :], cbuf)
                for g in range(_RC // 16):
                    rinv = 1.0 / degv[pl.ds(j * _RC + g * 16, 16)]
                    for r in range(16):
                        w = rinv[r]
                        for kk in range(NHID // 16):
                            fsl = pl.ds(kk * 16, 16)
                            ebuf0[g * 16 + r, fsl] = ebuf0[g * 16 + r, fsl] * w
                pltpu.sync_copy(cbuf, out_hbm.at[sl, :])
                return carry

            lax.fori_loop(0, _RPT // _RC, wout, 0)

        @pl.when(c == 0)
        def _():
            run_chunk(ga_hbm, sa_hbm)

        @pl.when(c == 1)
        def _():
            run_chunk(gb_hbm, sb_hbm)

    return k(ga, gb, src_rs, dst_rs, deg1d)


def _sc_hop_final(ga, gb, src_rs, dst_rs, d1, sel_rs):
    @functools.partial(
        pl.kernel,
        out_type=(jax.ShapeDtypeStruct((_SEL_PAD, NHID), jnp.float32),
                  jax.ShapeDtypeStruct((_SEL_PAD, NHID), jnp.float32),
                  jax.ShapeDtypeStruct((_SEL_PAD,), jnp.float32)),
        mesh=_MESH,
        scratch_types=_HOP_SCRATCH + [
            pltpu.VMEM((1, 4, _SB), jnp.int32),
            pltpu.VMEM((_SB,), jnp.float32),
        ],
    )
    def k(ga_hbm, gb_hbm, src_hbm, dst_hbm, d1_hbm, sel_hbm,
          oa_hbm, ob_hbm, o1_hbm,
          idxs_v, idxd_v, ebuf0, ebuf1, accum, sem0, sem1, sel_v, dbuf):
        cbuf = ebuf0.at[pl.ds(0, _RC), :]
        sbuf = ebuf0.at[pl.ds(0, _SB), :]
        c = lax.axis_index("c")
        s = lax.axis_index("s")

        def run_chunk(g_hbm, out_hbm):
            _hop_init(g_hbm, accum, cbuf, s)
            plsc.subcore_barrier()
            _hop_edges(g_hbm, src_hbm, dst_hbm, accum, idxs_v, idxd_v,
                       ebuf0, ebuf1, sem0, sem1, s)
            plsc.subcore_barrier()
            pltpu.sync_copy(sel_hbm.at[pl.ds(s, 1), :, :], sel_v)
            for j in range(4):
                base = s * (_SEL_PAD // _NTILE) + j * _SB
                pltpu.async_copy(accum.at[sel_v.at[0, j]], sbuf, sem0).wait()
                pltpu.sync_copy(sbuf, out_hbm.at[pl.ds(base, _SB), :])

        @pl.when(c == 0)
        def _():
            run_chunk(ga_hbm, oa_hbm)
            for j in range(4):
                base = s * (_SEL_PAD // _NTILE) + j * _SB
                pltpu.async_copy(d1_hbm.at[sel_v.at[0, j]], dbuf, sem0).wait()
                pltpu.sync_copy(dbuf, o1_hbm.at[pl.ds(base, _SB)])

        @pl.when(c == 1)
        def _():
            run_chunk(gb_hbm, ob_hbm)

    return k(ga, gb, src_rs, dst_rs, d1, sel_rs)


def kernel(x, edge_index, cluster_index, cluster_id, W1, b1, Wfc, bfc):
    src = edge_index[0].astype(jnp.int32)
    dst = edge_index[1].astype(jnp.int32)
    sel = cluster_index.astype(jnp.int32)
    src_rs = src.reshape(_NTILE, E // _NTILE // _EB, _EB)
    dst_rs = dst.reshape(_NTILE, E // _NTILE // _EB, _EB)

    hist = _sc_degree(dst)
    deg = hist + 1.0
    deg1d = jnp.pad(deg, (0, _NP - N), constant_values=1.0)

    sel_pad = jnp.pad(sel, (0, _SEL_PAD - NSEL))
    sel_rs = sel_pad.reshape(_NTILE, 4, _SB)

    ga, gb, d1 = _tc_scale_split(x, deg)
    ga1, gb1 = _sc_hop(ga, gb, src_rs, dst_rs, deg1d)
    oa, ob, o1 = _sc_hop_final(ga1, gb1, src_rs, dst_rs, d1, sel_rs)
    return _tc_tail(oa, ob, o1, cluster_id, W1, b1, Wfc, bfc)
